# baseline (speedup 1.0000x reference)
import functools

import jax
import jax.numpy as jnp
from jax import lax
from jax.experimental import pallas as pl
from jax.experimental.pallas import tpu as pltpu
from jax.experimental.pallas import tpu_sc as plsc

_PAD_BLK = 8192


def _pad_block(t_ref, o_ref):
    o_ref[:, : t_ref.shape[1]] = t_ref[...]
    o_ref[:, t_ref.shape[1] :] = jnp.zeros(
        (t_ref.shape[0], o_ref.shape[1] - t_ref.shape[1]), jnp.float32
    )


def kernel(image_ids, embeddings_weight):
    (B,) = image_ids.shape
    V, D = embeddings_weight.shape
    P = 128
    info = plsc.get_sparse_core_info()
    NC, NS = info.num_cores, info.num_subcores
    NW = NC * NS
    b_per_w = B // NW

    table128 = pl.pallas_call(
        _pad_block,
        grid=(pl.cdiv(V, _PAD_BLK),),
        in_specs=[pl.BlockSpec((_PAD_BLK, D), lambda i: (i, 0))],
        out_specs=pl.BlockSpec((_PAD_BLK, P), lambda i: (i, 0)),
        out_shape=jax.ShapeDtypeStruct((V, P), jnp.float32),
    )(embeddings_weight)

    mesh = plsc.VectorSubcoreMesh(core_axis_name="c", subcore_axis_name="s")

    @functools.partial(
        pl.kernel,
        mesh=mesh,
        out_type=jax.ShapeDtypeStruct((B, P), jnp.float32),
        scratch_types=[
            pltpu.VMEM((b_per_w,), jnp.int32),
            pltpu.VMEM((b_per_w, P), jnp.float32),
            pltpu.SemaphoreType.DMA,
        ],
    )
    def gather_kernel(idx_hbm, table_hbm, out_hbm, idx_v, rows_v, sem):
        wid = lax.axis_index("s") * NC + lax.axis_index("c")
        base = wid * b_per_w
        pltpu.sync_copy(idx_hbm.at[pl.ds(base, b_per_w)], idx_v)
        pltpu.async_copy(table_hbm.at[idx_v], rows_v, sem).wait()
        pltpu.sync_copy(rows_v, out_hbm.at[pl.ds(base, b_per_w)])

    out128 = gather_kernel(image_ids.astype(jnp.int32), table128)
    return out128[:, :D]


# speedup vs baseline: 1.7054x; 1.7054x over previous
import functools

import jax
import jax.numpy as jnp
from jax import lax
from jax.experimental import pallas as pl
from jax.experimental.pallas import tpu as pltpu
from jax.experimental.pallas import tpu_sc as plsc

_CHUNK = 16


def kernel(image_ids, embeddings_weight):
    (B,) = image_ids.shape
    V, D = embeddings_weight.shape
    info = plsc.get_sparse_core_info()
    NC, NS = info.num_cores, info.num_subcores
    NW = NC * NS
    assert B % (NW * _CHUNK) == 0
    b_per_w = B // NW
    half = b_per_w // 2
    n_chunks = half // _CHUNK

    mesh = plsc.VectorSubcoreMesh(core_axis_name="c", subcore_axis_name="s")

    @functools.partial(
        pl.kernel,
        mesh=mesh,
        out_type=jax.ShapeDtypeStruct((B, D), jnp.float32),
        scratch_types=[
            pltpu.VMEM((b_per_w,), jnp.int32),
            pltpu.VMEM((half, D), jnp.float32),
            pltpu.VMEM_SHARED((NS, half, D), jnp.float32),
            pltpu.SemaphoreType.DMA,
            pltpu.SemaphoreType.DMA,
        ],
    )
    def gather_kernel(idx_hbm, table_hbm, out_hbm, idx_v, rows_v, rows_sh,
                      sem_g, sem_s):
        sid = lax.axis_index("s")
        wid = sid * NC + lax.axis_index("c")
        base = wid * b_per_w
        pltpu.sync_copy(idx_hbm.at[pl.ds(base, b_per_w)], idx_v)

        @plsc.parallel_loop(0, n_chunks)
        def _fire(c):
            off = c * _CHUNK
            vec_a = idx_v[pl.ds(off, _CHUNK)]
            vec_b = idx_v[pl.ds(half + off, _CHUNK)]
            for j in range(_CHUNK):
                pltpu.async_copy(
                    table_hbm.at[pl.ds(vec_a[j], 1)],
                    rows_v.at[pl.ds(off, _CHUNK)].at[pl.ds(j, 1)],
                    sem_g,
                )
                pltpu.async_copy(
                    table_hbm.at[pl.ds(vec_b[j], 1)],
                    rows_sh.at[sid].at[pl.ds(off, _CHUNK)].at[pl.ds(j, 1)],
                    sem_s,
                )

        @pl.loop(0, n_chunks)
        def _drain(c):
            for j in range(_CHUNK):
                pltpu.make_async_copy(
                    table_hbm.at[pl.ds(0, 1)],
                    rows_v.at[pl.ds(0, _CHUNK)].at[pl.ds(j, 1)],
                    sem_g,
                ).wait()
                pltpu.make_async_copy(
                    table_hbm.at[pl.ds(0, 1)],
                    rows_sh.at[0].at[pl.ds(0, _CHUNK)].at[pl.ds(j, 1)],
                    sem_s,
                ).wait()

        pltpu.sync_copy(rows_v, out_hbm.at[pl.ds(base, half)])
        pltpu.sync_copy(rows_sh.at[sid], out_hbm.at[pl.ds(base + half, half)])

    return gather_kernel(image_ids.astype(jnp.int32), embeddings_weight)


# speedup vs baseline: 1.7646x; 1.0347x over previous
import functools

import jax
import jax.numpy as jnp
from jax import lax
from jax.experimental import pallas as pl
from jax.experimental.pallas import tpu as pltpu
from jax.experimental.pallas import tpu_sc as plsc

_CHUNK = 16


def kernel(image_ids, embeddings_weight):
    (B,) = image_ids.shape
    V, D = embeddings_weight.shape
    info = plsc.get_sparse_core_info()
    NC, NS = info.num_cores, info.num_subcores
    NW = NC * NS
    assert B % (NW * _CHUNK) == 0
    b_per_w = B // NW
    n_chunks = b_per_w // _CHUNK

    mesh = plsc.VectorSubcoreMesh(core_axis_name="c", subcore_axis_name="s")

    @functools.partial(
        pl.kernel,
        mesh=mesh,
        out_type=jax.ShapeDtypeStruct((B, D), jnp.float32),
        scratch_types=[
            pltpu.VMEM((b_per_w,), jnp.int32),
            pltpu.VMEM((b_per_w, D), jnp.float32),
            pltpu.SemaphoreType.DMA,
        ],
    )
    def gather_kernel(idx_hbm, table_hbm, out_hbm, idx_v, rows_v, sem):
        wid = lax.axis_index("s") * NC + lax.axis_index("c")
        base = wid * b_per_w
        with jax.named_scope("stage_idx"):
            pltpu.sync_copy(idx_hbm.at[pl.ds(base, b_per_w)], idx_v)

        with jax.named_scope("fire_gathers"):

            @plsc.parallel_loop(0, n_chunks)
            def _fire(c):
                off = c * _CHUNK
                idx_vec = idx_v[pl.ds(off, _CHUNK)]
                for j in range(_CHUNK):
                    pltpu.async_copy(
                        table_hbm.at[pl.ds(idx_vec[j], 1)],
                        rows_v.at[pl.ds(off, _CHUNK)].at[pl.ds(j, 1)],
                        sem,
                    )

        with jax.named_scope("drain_gathers"):

            @pl.loop(0, n_chunks)
            def _drain(c):
                for j in range(_CHUNK):
                    pltpu.make_async_copy(
                        table_hbm.at[pl.ds(0, 1)],
                        rows_v.at[pl.ds(0, _CHUNK)].at[pl.ds(j, 1)],
                        sem,
                    ).wait()

        with jax.named_scope("writeback"):
            pltpu.sync_copy(rows_v, out_hbm.at[pl.ds(base, b_per_w)])

    return gather_kernel(image_ids.astype(jnp.int32), embeddings_weight)
